# baseline (speedup 1.0000x reference)
import jax
import jax.numpy as jnp
from jax.experimental import pallas as pl
from jax.experimental.pallas import tpu as pltpu


def _round_up(x, m):
    return (x + m - 1) // m * m


def _make_stream_kernel(bm, nsteps, nk, tkw):
    hm = bm // 2

    def body(xa_ref, xb_ref, w_ref, s_ref, t_ref, o_ref, wres, wsem):
        i = pl.program_id(0)

        def w_copy(k):
            return pltpu.make_async_copy(
                w_ref.at[pl.ds(k * tkw, tkw), :],
                wres.at[pl.ds(k * tkw, tkw), :], wsem.at[k])

        @pl.when(i == 0)
        def _first_step():
            for k in range(nk):
                w_copy(k).start()
            for k in range(nk):
                w_copy(k).wait()
                wk = wres[k * tkw:(k + 1) * tkw, :]
                pa = jnp.dot(xa_ref[:, k * tkw:(k + 1) * tkw], wk,
                             preferred_element_type=jnp.float32)
                pb = jnp.dot(xb_ref[:, k * tkw:(k + 1) * tkw], wk,
                             preferred_element_type=jnp.float32)
                if k == 0:
                    o_ref[:hm, :] = pa
                    o_ref[hm:, :] = pb
                else:
                    o_ref[:hm, :] += pa
                    o_ref[hm:, :] += pb
            y = o_ref[...] * s_ref[...] + t_ref[...]
            o_ref[...] = jnp.maximum(y, 0.0)

        @pl.when(i > 0)
        def _steady():
            wv = wres[...]
            acc_a = jnp.dot(xa_ref[...], wv,
                            preferred_element_type=jnp.float32)
            acc_b = jnp.dot(xb_ref[...], wv,
                            preferred_element_type=jnp.float32)
            o_ref[:hm, :] = jnp.maximum(
                acc_a * s_ref[...] + t_ref[...], 0.0)
            o_ref[hm:, :] = jnp.maximum(
                acc_b * s_ref[...] + t_ref[...], 0.0)

    return body


def _stream_linear_bn_relu(x2d, w_t, s2, t2, *, bm, nk):
    M, Din = x2d.shape
    Dout = w_t.shape[1]
    nsteps = M // bm
    tkw = Din // nk
    hm = bm // 2

    flops = 2 * M * Din * Dout
    bytes_accessed = M * Din * 4 + Din * Dout * 4 + M * Dout * 4
    cost = pl.CostEstimate(flops=flops, transcendentals=0,
                           bytes_accessed=bytes_accessed)

    return pl.pallas_call(
        _make_stream_kernel(bm, nsteps, nk, tkw),
        grid=(nsteps,),
        out_shape=jax.ShapeDtypeStruct((M, Dout), x2d.dtype),
        in_specs=[
            pl.BlockSpec((hm, Din), lambda i: (2 * i, 0)),
            pl.BlockSpec((hm, Din), lambda i: (2 * i + 1, 0)),
            pl.BlockSpec(memory_space=pl.ANY),
            pl.BlockSpec((1, Dout), lambda i: (0, 0)),
            pl.BlockSpec((1, Dout), lambda i: (0, 0)),
        ],
        out_specs=pl.BlockSpec((bm, Dout), lambda i: (i, 0)),
        scratch_shapes=[
            pltpu.VMEM((Din, Dout), jnp.float32),
            pltpu.SemaphoreType.DMA((nk,)),
        ],
        compiler_params=pltpu.CompilerParams(
            dimension_semantics=("arbitrary",),
            vmem_limit_bytes=100 * 1024 * 1024,
        ),
        cost_estimate=cost,
    )(x2d, x2d, w_t, s2, t2)


def _fused_rowblock_kernel(x_ref, w_ref, s_ref, t_ref, o_ref):
    acc = jnp.dot(x_ref[...], w_ref[...], preferred_element_type=jnp.float32)
    y = acc * s_ref[...] + t_ref[...]
    o_ref[...] = jnp.maximum(y, 0.0).astype(o_ref.dtype)


def _blockspec_linear_bn_relu(x2d, w_t, s2, t2, *, bm=512):
    M, Din = x2d.shape
    Dout = w_t.shape[1]

    bm = min(bm, _round_up(M, 8))
    Mp = _round_up(M, bm)
    if Mp != M:
        x2d = jnp.pad(x2d, ((0, Mp - M), (0, 0)))
    nsteps = Mp // bm

    flops = 2 * Mp * Din * Dout
    bytes_accessed = Mp * Din * 4 + Din * Dout * 4 + Mp * Dout * 4
    cost = pl.CostEstimate(flops=flops, transcendentals=0,
                           bytes_accessed=bytes_accessed)

    out = pl.pallas_call(
        _fused_rowblock_kernel,
        grid=(nsteps,),
        out_shape=jax.ShapeDtypeStruct((Mp, Dout), x2d.dtype),
        in_specs=[
            pl.BlockSpec((bm, Din), lambda j: (j, 0)),
            pl.BlockSpec((Din, Dout), lambda j: (0, 0)),
            pl.BlockSpec((1, Dout), lambda j: (0, 0)),
            pl.BlockSpec((1, Dout), lambda j: (0, 0)),
        ],
        out_specs=pl.BlockSpec((bm, Dout), lambda j: (j, 0)),
        compiler_params=pltpu.CompilerParams(
            dimension_semantics=("arbitrary",),
            vmem_limit_bytes=100 * 1024 * 1024,
        ),
        cost_estimate=cost,
    )(x2d, w_t, s2, t2)

    return out[:M] if Mp != M else out


def _fused_linear_bn_relu(x2d, w_t, scale, shift, *, bm=1024, nk=4):
    M, Din = x2d.shape
    Dout = w_t.shape[1]
    s2 = scale.reshape(1, Dout).astype(jnp.float32)
    t2 = shift.reshape(1, Dout).astype(jnp.float32)

    if (M % bm == 0 and Din % nk == 0 and (bm // 2) % 8 == 0
            and (Din // nk) % 8 == 0 and Dout % 128 == 0):
        return _stream_linear_bn_relu(x2d, w_t, s2, t2, bm=bm, nk=nk)
    return _blockspec_linear_bn_relu(x2d, w_t, s2, t2)


def kernel(x, w_t, b, bn_gamma, bn_beta, bn_mean, bn_var):
    eps = 1e-5
    s = bn_gamma * jax.lax.rsqrt(bn_var + eps)
    t = (b - bn_mean) * s + bn_beta

    if x.ndim == 3:
        N, K, Din = x.shape
        y = _fused_linear_bn_relu(x.reshape(N * K, Din), w_t, s, t)
        return y.reshape(N, K, -1)
    return _fused_linear_bn_relu(x, w_t, s, t)


# speedup vs baseline: 1.0377x; 1.0377x over previous
import jax
import jax.numpy as jnp
from jax.experimental import pallas as pl
from jax.experimental.pallas import tpu as pltpu


def _round_up(x, m):
    return (x + m - 1) // m * m


def _make_stream_kernel(bm, nsteps, nk, tkw):
    hm = bm // 2

    def body(xa_ref, xb_ref, w_ref, s_ref, t_ref, o_ref,
             wf32, wb, xb16, wsem):
        i = pl.program_id(0)

        def w_copy(k):
            return pltpu.make_async_copy(
                w_ref.at[pl.ds(k * tkw, tkw), :], wf32.at[k], wsem.at[k])

        xb16[:hm, :] = xa_ref[...].astype(jnp.bfloat16)
        xb16[hm:, :] = xb_ref[...].astype(jnp.bfloat16)

        @pl.when(i == 0)
        def _first_step():
            for k in range(nk):
                w_copy(k).start()
            for k in range(nk):
                w_copy(k).wait()
                wb[pl.ds(k * tkw, tkw), :] = wf32[k].astype(jnp.bfloat16)
                p = jnp.dot(xb16[:, k * tkw:(k + 1) * tkw],
                            wb[k * tkw:(k + 1) * tkw, :],
                            preferred_element_type=jnp.float32)
                if k == 0:
                    o_ref[...] = p
                else:
                    o_ref[...] += p
            y = o_ref[...] * s_ref[...] + t_ref[...]
            o_ref[...] = jnp.maximum(y, 0.0)

        @pl.when(i > 0)
        def _steady():
            acc = jnp.dot(xb16[...], wb[...],
                          preferred_element_type=jnp.float32)
            o_ref[...] = jnp.maximum(acc * s_ref[...] + t_ref[...], 0.0)

    return body


def _stream_linear_bn_relu(x2d, w_t, s2, t2, *, bm, nk):
    M, Din = x2d.shape
    Dout = w_t.shape[1]
    nsteps = M // bm
    tkw = Din // nk
    hm = bm // 2

    flops = 2 * M * Din * Dout
    bytes_accessed = M * Din * 4 + Din * Dout * 4 + M * Dout * 4
    cost = pl.CostEstimate(flops=flops, transcendentals=0,
                           bytes_accessed=bytes_accessed)

    return pl.pallas_call(
        _make_stream_kernel(bm, nsteps, nk, tkw),
        grid=(nsteps,),
        out_shape=jax.ShapeDtypeStruct((M, Dout), x2d.dtype),
        in_specs=[
            pl.BlockSpec((hm, Din), lambda i: (2 * i, 0)),
            pl.BlockSpec((hm, Din), lambda i: (2 * i + 1, 0)),
            pl.BlockSpec(memory_space=pl.ANY),
            pl.BlockSpec((1, Dout), lambda i: (0, 0)),
            pl.BlockSpec((1, Dout), lambda i: (0, 0)),
        ],
        out_specs=pl.BlockSpec((bm, Dout), lambda i: (i, 0)),
        scratch_shapes=[
            pltpu.VMEM((nk, tkw, Dout), jnp.float32),
            pltpu.VMEM((Din, Dout), jnp.bfloat16),
            pltpu.VMEM((bm, Din), jnp.bfloat16),
            pltpu.SemaphoreType.DMA((nk,)),
        ],
        compiler_params=pltpu.CompilerParams(
            dimension_semantics=("arbitrary",),
            vmem_limit_bytes=100 * 1024 * 1024,
        ),
        cost_estimate=cost,
    )(x2d, x2d, w_t, s2, t2)


def _fused_rowblock_kernel(x_ref, w_ref, s_ref, t_ref, o_ref, wb_ref, xb_ref):
    j = pl.program_id(0)

    @pl.when(j == 0)
    def _():
        wb_ref[...] = w_ref[...].astype(jnp.bfloat16)

    xb_ref[...] = x_ref[...].astype(jnp.bfloat16)
    acc = jnp.dot(xb_ref[...], wb_ref[...],
                  preferred_element_type=jnp.float32)
    y = acc * s_ref[...] + t_ref[...]
    o_ref[...] = jnp.maximum(y, 0.0).astype(o_ref.dtype)


def _blockspec_linear_bn_relu(x2d, w_t, s2, t2, *, bm=512):
    M, Din = x2d.shape
    Dout = w_t.shape[1]

    bm = min(bm, _round_up(M, 8))
    Mp = _round_up(M, bm)
    if Mp != M:
        x2d = jnp.pad(x2d, ((0, Mp - M), (0, 0)))
    nsteps = Mp // bm

    flops = 2 * Mp * Din * Dout
    bytes_accessed = Mp * Din * 4 + Din * Dout * 4 + Mp * Dout * 4
    cost = pl.CostEstimate(flops=flops, transcendentals=0,
                           bytes_accessed=bytes_accessed)

    out = pl.pallas_call(
        _fused_rowblock_kernel,
        grid=(nsteps,),
        out_shape=jax.ShapeDtypeStruct((Mp, Dout), x2d.dtype),
        in_specs=[
            pl.BlockSpec((bm, Din), lambda j: (j, 0)),
            pl.BlockSpec((Din, Dout), lambda j: (0, 0)),
            pl.BlockSpec((1, Dout), lambda j: (0, 0)),
            pl.BlockSpec((1, Dout), lambda j: (0, 0)),
        ],
        out_specs=pl.BlockSpec((bm, Dout), lambda j: (j, 0)),
        scratch_shapes=[
            pltpu.VMEM((Din, Dout), jnp.bfloat16),
            pltpu.VMEM((bm, Din), jnp.bfloat16),
        ],
        compiler_params=pltpu.CompilerParams(
            dimension_semantics=("arbitrary",),
            vmem_limit_bytes=100 * 1024 * 1024,
        ),
        cost_estimate=cost,
    )(x2d, w_t, s2, t2)

    return out[:M] if Mp != M else out


def _fused_linear_bn_relu(x2d, w_t, scale, shift, *, bm=512, nk=4):
    M, Din = x2d.shape
    Dout = w_t.shape[1]
    s2 = scale.reshape(1, Dout).astype(jnp.float32)
    t2 = shift.reshape(1, Dout).astype(jnp.float32)

    if (M % bm == 0 and Din % nk == 0 and (bm // 2) % 8 == 0
            and (Din // nk) % 8 == 0 and Dout % 128 == 0):
        return _stream_linear_bn_relu(x2d, w_t, s2, t2, bm=bm, nk=nk)
    return _blockspec_linear_bn_relu(x2d, w_t, s2, t2)


def kernel(x, w_t, b, bn_gamma, bn_beta, bn_mean, bn_var):
    eps = 1e-5
    s = bn_gamma * jax.lax.rsqrt(bn_var + eps)
    t = (b - bn_mean) * s + bn_beta

    if x.ndim == 3:
        N, K, Din = x.shape
        y = _fused_linear_bn_relu(x.reshape(N * K, Din), w_t, s, t)
        return y.reshape(N, K, -1)
    return _fused_linear_bn_relu(x, w_t, s, t)


# speedup vs baseline: 1.0774x; 1.0382x over previous
import jax
import jax.numpy as jnp
from jax.experimental import pallas as pl
from jax.experimental.pallas import tpu as pltpu


def _round_up(x, m):
    return (x + m - 1) // m * m


def _make_stream_kernel(bm, nsteps, nk, tkw):
    hm = bm // 2

    def body(xa_ref, xb_ref, w_ref, s_ref, t_ref, o_ref, wres, wsem):
        i = pl.program_id(0)

        def w_copy(k):
            return pltpu.make_async_copy(
                w_ref.at[pl.ds(k * tkw, tkw), :],
                wres.at[pl.ds(k * tkw, tkw), :], wsem.at[k])

        @pl.when(i == 0)
        def _first_step():
            for k in range(nk):
                w_copy(k).start()
            for k in range(nk):
                w_copy(k).wait()
                wk = wres[k * tkw:(k + 1) * tkw, :]
                pa = jnp.dot(xa_ref[:, k * tkw:(k + 1) * tkw], wk,
                             preferred_element_type=jnp.float32)
                pb = jnp.dot(xb_ref[:, k * tkw:(k + 1) * tkw], wk,
                             preferred_element_type=jnp.float32)
                if k == 0:
                    o_ref[:hm, :] = pa
                    o_ref[hm:, :] = pb
                else:
                    o_ref[:hm, :] += pa
                    o_ref[hm:, :] += pb
            y = o_ref[...] * s_ref[...] + t_ref[...]
            o_ref[...] = jnp.maximum(y, 0.0)

        @pl.when(i > 0)
        def _steady():
            wv = wres[...]
            acc_a = jnp.dot(xa_ref[...], wv,
                            preferred_element_type=jnp.float32)
            acc_b = jnp.dot(xb_ref[...], wv,
                            preferred_element_type=jnp.float32)
            o_ref[:hm, :] = jnp.maximum(
                acc_a * s_ref[...] + t_ref[...], 0.0)
            o_ref[hm:, :] = jnp.maximum(
                acc_b * s_ref[...] + t_ref[...], 0.0)

    return body


def _stream_linear_bn_relu(x2d, w_t, s2, t2, *, bm, nk):
    M, Din = x2d.shape
    Dout = w_t.shape[1]
    nsteps = M // bm
    tkw = Din // nk
    hm = bm // 2

    flops = 2 * M * Din * Dout
    bytes_accessed = M * Din * 4 + Din * Dout * 4 + M * Dout * 4
    cost = pl.CostEstimate(flops=flops, transcendentals=0,
                           bytes_accessed=bytes_accessed)

    return pl.pallas_call(
        _make_stream_kernel(bm, nsteps, nk, tkw),
        grid=(nsteps,),
        out_shape=jax.ShapeDtypeStruct((M, Dout), x2d.dtype),
        in_specs=[
            pl.BlockSpec((hm, Din), lambda i: (2 * i, 0)),
            pl.BlockSpec((hm, Din), lambda i: (2 * i + 1, 0)),
            pl.BlockSpec(memory_space=pl.ANY),
            pl.BlockSpec((1, Dout), lambda i: (0, 0)),
            pl.BlockSpec((1, Dout), lambda i: (0, 0)),
        ],
        out_specs=pl.BlockSpec((bm, Dout), lambda i: (i, 0)),
        scratch_shapes=[
            pltpu.VMEM((Din, Dout), jnp.float32),
            pltpu.SemaphoreType.DMA((nk,)),
        ],
        compiler_params=pltpu.CompilerParams(
            dimension_semantics=("arbitrary",),
            vmem_limit_bytes=100 * 1024 * 1024,
        ),
        cost_estimate=cost,
    )(x2d, x2d, w_t, s2, t2)


def _fused_rowblock_kernel(x_ref, w_ref, s_ref, t_ref, o_ref):
    acc = jnp.dot(x_ref[...], w_ref[...], preferred_element_type=jnp.float32)
    y = acc * s_ref[...] + t_ref[...]
    o_ref[...] = jnp.maximum(y, 0.0).astype(o_ref.dtype)


def _blockspec_linear_bn_relu(x2d, w_t, s2, t2, *, bm=512):
    M, Din = x2d.shape
    Dout = w_t.shape[1]

    bm = min(bm, _round_up(M, 8))
    Mp = _round_up(M, bm)
    if Mp != M:
        x2d = jnp.pad(x2d, ((0, Mp - M), (0, 0)))
    nsteps = Mp // bm

    flops = 2 * Mp * Din * Dout
    bytes_accessed = Mp * Din * 4 + Din * Dout * 4 + Mp * Dout * 4
    cost = pl.CostEstimate(flops=flops, transcendentals=0,
                           bytes_accessed=bytes_accessed)

    out = pl.pallas_call(
        _fused_rowblock_kernel,
        grid=(nsteps,),
        out_shape=jax.ShapeDtypeStruct((Mp, Dout), x2d.dtype),
        in_specs=[
            pl.BlockSpec((bm, Din), lambda j: (j, 0)),
            pl.BlockSpec((Din, Dout), lambda j: (0, 0)),
            pl.BlockSpec((1, Dout), lambda j: (0, 0)),
            pl.BlockSpec((1, Dout), lambda j: (0, 0)),
        ],
        out_specs=pl.BlockSpec((bm, Dout), lambda j: (j, 0)),
        compiler_params=pltpu.CompilerParams(
            dimension_semantics=("arbitrary",),
            vmem_limit_bytes=100 * 1024 * 1024,
        ),
        cost_estimate=cost,
    )(x2d, w_t, s2, t2)

    return out[:M] if Mp != M else out


def _fused_linear_bn_relu(x2d, w_t, scale, shift, *, bm=512, nk=8):
    M, Din = x2d.shape
    Dout = w_t.shape[1]
    s2 = scale.reshape(1, Dout).astype(jnp.float32)
    t2 = shift.reshape(1, Dout).astype(jnp.float32)

    if (M % bm == 0 and Din % nk == 0 and (bm // 2) % 8 == 0
            and (Din // nk) % 8 == 0 and Dout % 128 == 0):
        return _stream_linear_bn_relu(x2d, w_t, s2, t2, bm=bm, nk=nk)
    return _blockspec_linear_bn_relu(x2d, w_t, s2, t2)


def kernel(x, w_t, b, bn_gamma, bn_beta, bn_mean, bn_var):
    eps = 1e-5
    s = bn_gamma * jax.lax.rsqrt(bn_var + eps)
    t = (b - bn_mean) * s + bn_beta

    if x.ndim == 3:
        N, K, Din = x.shape
        y = _fused_linear_bn_relu(x.reshape(N * K, Din), w_t, s, t)
        return y.reshape(N, K, -1)
    return _fused_linear_bn_relu(x, w_t, s, t)


# speedup vs baseline: 1.0808x; 1.0032x over previous
import jax
import jax.numpy as jnp
from jax.experimental import pallas as pl
from jax.experimental.pallas import tpu as pltpu


def _round_up(x, m):
    return (x + m - 1) // m * m


def _make_stream_kernel(bm, nsteps, nk, tkw):
    qm = bm // 4

    def body(xa_ref, xb_ref, xc_ref, xd_ref, w_ref, s_ref, t_ref, o_ref,
             wres, wsem):
        i = pl.program_id(0)

        def w_copy(k):
            return pltpu.make_async_copy(
                w_ref.at[pl.ds(k * tkw, tkw), :],
                wres.at[pl.ds(k * tkw, tkw), :], wsem.at[k])

        xrefs = (xa_ref, xb_ref, xc_ref, xd_ref)

        @pl.when(i == 0)
        def _first_step():
            for k in range(nk):
                w_copy(k).start()
            for k in range(nk):
                w_copy(k).wait()
                wk = wres[k * tkw:(k + 1) * tkw, :]
                for h, xr in enumerate(xrefs):
                    p = jnp.dot(xr[:, k * tkw:(k + 1) * tkw], wk,
                                preferred_element_type=jnp.float32)
                    if k == 0:
                        o_ref[h * qm:(h + 1) * qm, :] = p
                    else:
                        o_ref[h * qm:(h + 1) * qm, :] += p
            y = o_ref[...] * s_ref[...] + t_ref[...]
            o_ref[...] = jnp.maximum(y, 0.0)

        @pl.when(i > 0)
        def _steady():
            wv = wres[...]
            for h, xr in enumerate(xrefs):
                acc = jnp.dot(xr[...], wv,
                              preferred_element_type=jnp.float32)
                o_ref[h * qm:(h + 1) * qm, :] = jnp.maximum(
                    acc * s_ref[...] + t_ref[...], 0.0)

    return body


def _stream_linear_bn_relu(x2d, w_t, s2, t2, *, bm, nk):
    M, Din = x2d.shape
    Dout = w_t.shape[1]
    nsteps = M // bm
    tkw = Din // nk
    qm = bm // 4

    flops = 2 * M * Din * Dout
    bytes_accessed = M * Din * 4 + Din * Dout * 4 + M * Dout * 4
    cost = pl.CostEstimate(flops=flops, transcendentals=0,
                           bytes_accessed=bytes_accessed)

    return pl.pallas_call(
        _make_stream_kernel(bm, nsteps, nk, tkw),
        grid=(nsteps,),
        out_shape=jax.ShapeDtypeStruct((M, Dout), x2d.dtype),
        in_specs=[
            pl.BlockSpec((qm, Din), lambda i: (4 * i, 0)),
            pl.BlockSpec((qm, Din), lambda i: (4 * i + 1, 0)),
            pl.BlockSpec((qm, Din), lambda i: (4 * i + 2, 0)),
            pl.BlockSpec((qm, Din), lambda i: (4 * i + 3, 0)),
            pl.BlockSpec(memory_space=pl.ANY),
            pl.BlockSpec((1, Dout), lambda i: (0, 0)),
            pl.BlockSpec((1, Dout), lambda i: (0, 0)),
        ],
        out_specs=pl.BlockSpec((bm, Dout), lambda i: (i, 0)),
        scratch_shapes=[
            pltpu.VMEM((Din, Dout), jnp.float32),
            pltpu.SemaphoreType.DMA((nk,)),
        ],
        compiler_params=pltpu.CompilerParams(
            dimension_semantics=("arbitrary",),
            vmem_limit_bytes=100 * 1024 * 1024,
        ),
        cost_estimate=cost,
    )(x2d, x2d, x2d, x2d, w_t, s2, t2)


def _fused_rowblock_kernel(x_ref, w_ref, s_ref, t_ref, o_ref):
    acc = jnp.dot(x_ref[...], w_ref[...], preferred_element_type=jnp.float32)
    y = acc * s_ref[...] + t_ref[...]
    o_ref[...] = jnp.maximum(y, 0.0).astype(o_ref.dtype)


def _blockspec_linear_bn_relu(x2d, w_t, s2, t2, *, bm=512):
    M, Din = x2d.shape
    Dout = w_t.shape[1]

    bm = min(bm, _round_up(M, 8))
    Mp = _round_up(M, bm)
    if Mp != M:
        x2d = jnp.pad(x2d, ((0, Mp - M), (0, 0)))
    nsteps = Mp // bm

    flops = 2 * Mp * Din * Dout
    bytes_accessed = Mp * Din * 4 + Din * Dout * 4 + Mp * Dout * 4
    cost = pl.CostEstimate(flops=flops, transcendentals=0,
                           bytes_accessed=bytes_accessed)

    out = pl.pallas_call(
        _fused_rowblock_kernel,
        grid=(nsteps,),
        out_shape=jax.ShapeDtypeStruct((Mp, Dout), x2d.dtype),
        in_specs=[
            pl.BlockSpec((bm, Din), lambda j: (j, 0)),
            pl.BlockSpec((Din, Dout), lambda j: (0, 0)),
            pl.BlockSpec((1, Dout), lambda j: (0, 0)),
            pl.BlockSpec((1, Dout), lambda j: (0, 0)),
        ],
        out_specs=pl.BlockSpec((bm, Dout), lambda j: (j, 0)),
        compiler_params=pltpu.CompilerParams(
            dimension_semantics=("arbitrary",),
            vmem_limit_bytes=100 * 1024 * 1024,
        ),
        cost_estimate=cost,
    )(x2d, w_t, s2, t2)

    return out[:M] if Mp != M else out


def _fused_linear_bn_relu(x2d, w_t, scale, shift, *, bm=512, nk=8):
    M, Din = x2d.shape
    Dout = w_t.shape[1]
    s2 = scale.reshape(1, Dout).astype(jnp.float32)
    t2 = shift.reshape(1, Dout).astype(jnp.float32)

    if (M % bm == 0 and Din % nk == 0 and (bm // 4) % 8 == 0
            and (Din // nk) % 8 == 0 and Dout % 128 == 0):
        return _stream_linear_bn_relu(x2d, w_t, s2, t2, bm=bm, nk=nk)
    return _blockspec_linear_bn_relu(x2d, w_t, s2, t2)


def kernel(x, w_t, b, bn_gamma, bn_beta, bn_mean, bn_var):
    eps = 1e-5
    s = bn_gamma * jax.lax.rsqrt(bn_var + eps)
    t = (b - bn_mean) * s + bn_beta

    if x.ndim == 3:
        N, K, Din = x.shape
        y = _fused_linear_bn_relu(x.reshape(N * K, Din), w_t, s, t)
        return y.reshape(N, K, -1)
    return _fused_linear_bn_relu(x, w_t, s, t)


# speedup vs baseline: 1.0824x; 1.0015x over previous
import jax
import jax.numpy as jnp
from jax.experimental import pallas as pl
from jax.experimental.pallas import tpu as pltpu


def _round_up(x, m):
    return (x + m - 1) // m * m


def _make_stream_kernel(bm, nsteps, nk, tkw):
    hm = bm // 2

    def body(xa_ref, xb_ref, w_ref, s_ref, t_ref, o_ref, wres, wsem):
        i = pl.program_id(0)

        def w_copy(k):
            return pltpu.make_async_copy(
                w_ref.at[pl.ds(k * tkw, tkw), :],
                wres.at[pl.ds(k * tkw, tkw), :], wsem.at[k])

        @pl.when(i == 0)
        def _first_step():
            for k in range(nk):
                w_copy(k).start()
            for k in range(nk):
                w_copy(k).wait()
                wk = wres[k * tkw:(k + 1) * tkw, :]
                pa = jnp.dot(xa_ref[:, k * tkw:(k + 1) * tkw], wk,
                             preferred_element_type=jnp.float32)
                pb = jnp.dot(xb_ref[:, k * tkw:(k + 1) * tkw], wk,
                             preferred_element_type=jnp.float32)
                if k == 0:
                    o_ref[:hm, :] = pa
                    o_ref[hm:, :] = pb
                else:
                    o_ref[:hm, :] += pa
                    o_ref[hm:, :] += pb
            y = o_ref[...] * s_ref[...] + t_ref[...]
            o_ref[...] = jnp.maximum(y, 0.0)

        @pl.when(i > 0)
        def _steady():
            wv = wres[...]
            acc_a = jnp.dot(xa_ref[...], wv,
                            preferred_element_type=jnp.float32)
            acc_b = jnp.dot(xb_ref[...], wv,
                            preferred_element_type=jnp.float32)
            o_ref[:hm, :] = jnp.maximum(
                acc_a * s_ref[...] + t_ref[...], 0.0)
            o_ref[hm:, :] = jnp.maximum(
                acc_b * s_ref[...] + t_ref[...], 0.0)

    return body


def _stream_linear_bn_relu(x2d, w_t, s2, t2, *, bm, nk):
    M, Din = x2d.shape
    Dout = w_t.shape[1]
    nsteps = M // bm
    tkw = Din // nk
    hm = bm // 2

    flops = 2 * M * Din * Dout
    bytes_accessed = M * Din * 4 + Din * Dout * 4 + M * Dout * 4
    cost = pl.CostEstimate(flops=flops, transcendentals=0,
                           bytes_accessed=bytes_accessed)

    return pl.pallas_call(
        _make_stream_kernel(bm, nsteps, nk, tkw),
        grid=(nsteps,),
        out_shape=jax.ShapeDtypeStruct((M, Dout), x2d.dtype),
        in_specs=[
            pl.BlockSpec((hm, Din), lambda i: (2 * i, 0)),
            pl.BlockSpec((hm, Din), lambda i: (2 * i + 1, 0)),
            pl.BlockSpec(memory_space=pl.ANY),
            pl.BlockSpec((1, Dout), lambda i: (0, 0)),
            pl.BlockSpec((1, Dout), lambda i: (0, 0)),
        ],
        out_specs=pl.BlockSpec((bm, Dout), lambda i: (i, 0)),
        scratch_shapes=[
            pltpu.VMEM((Din, Dout), jnp.float32),
            pltpu.SemaphoreType.DMA((nk,)),
        ],
        compiler_params=pltpu.CompilerParams(
            dimension_semantics=("arbitrary",),
            vmem_limit_bytes=100 * 1024 * 1024,
        ),
        cost_estimate=cost,
    )(x2d, x2d, w_t, s2, t2)


def _fused_rowblock_kernel(x_ref, w_ref, s_ref, t_ref, o_ref):
    acc = jnp.dot(x_ref[...], w_ref[...], preferred_element_type=jnp.float32)
    y = acc * s_ref[...] + t_ref[...]
    o_ref[...] = jnp.maximum(y, 0.0).astype(o_ref.dtype)


def _blockspec_linear_bn_relu(x2d, w_t, s2, t2, *, bm=512):
    M, Din = x2d.shape
    Dout = w_t.shape[1]

    bm = min(bm, _round_up(M, 8))
    Mp = _round_up(M, bm)
    if Mp != M:
        x2d = jnp.pad(x2d, ((0, Mp - M), (0, 0)))
    nsteps = Mp // bm

    flops = 2 * Mp * Din * Dout
    bytes_accessed = Mp * Din * 4 + Din * Dout * 4 + Mp * Dout * 4
    cost = pl.CostEstimate(flops=flops, transcendentals=0,
                           bytes_accessed=bytes_accessed)

    out = pl.pallas_call(
        _fused_rowblock_kernel,
        grid=(nsteps,),
        out_shape=jax.ShapeDtypeStruct((Mp, Dout), x2d.dtype),
        in_specs=[
            pl.BlockSpec((bm, Din), lambda j: (j, 0)),
            pl.BlockSpec((Din, Dout), lambda j: (0, 0)),
            pl.BlockSpec((1, Dout), lambda j: (0, 0)),
            pl.BlockSpec((1, Dout), lambda j: (0, 0)),
        ],
        out_specs=pl.BlockSpec((bm, Dout), lambda j: (j, 0)),
        compiler_params=pltpu.CompilerParams(
            dimension_semantics=("arbitrary",),
            vmem_limit_bytes=100 * 1024 * 1024,
        ),
        cost_estimate=cost,
    )(x2d, w_t, s2, t2)

    return out[:M] if Mp != M else out


def _fused_linear_bn_relu(x2d, w_t, scale, shift, *, bm=512, nk=8):
    M, Din = x2d.shape
    Dout = w_t.shape[1]
    s2 = scale.reshape(1, Dout).astype(jnp.float32)
    t2 = shift.reshape(1, Dout).astype(jnp.float32)

    if (M % bm == 0 and Din % nk == 0 and (bm // 2) % 8 == 0
            and (Din // nk) % 8 == 0 and Dout % 128 == 0):
        return _stream_linear_bn_relu(x2d, w_t, s2, t2, bm=bm, nk=nk)
    return _blockspec_linear_bn_relu(x2d, w_t, s2, t2)


def kernel(x, w_t, b, bn_gamma, bn_beta, bn_mean, bn_var):
    eps = 1e-5
    s = bn_gamma * jax.lax.rsqrt(bn_var + eps)
    t = (b - bn_mean) * s + bn_beta

    if x.ndim == 3:
        N, K, Din = x.shape
        y = _fused_linear_bn_relu(x.reshape(N * K, Din), w_t, s, t)
        return y.reshape(N, K, -1)
    return _fused_linear_bn_relu(x, w_t, s, t)
